# Initial kernel scaffold; baseline (speedup 1.0000x reference)
import jax
import jax.numpy as jnp
from jax.experimental import pallas as pl


def kernel(x, weight):
    raise NotImplementedError("write your pallas kernel here")


# speedup vs baseline: 1.7955x; 1.7955x over previous
import functools

import jax
import jax.numpy as jnp
from jax import lax
from jax.experimental import pallas as pl
from jax.experimental.pallas import tpu as pltpu
from jax.experimental.pallas import tpu_sc as plsc

NC = 2
NS = 16
NW = NC * NS

IDX_VEC = 128
CH = 512
G = CH // IDX_VEC


@functools.partial(jax.jit, static_argnums=(2, 3))
def _emb_call(xf, weight, n, d):
    n_per_w = n // NW
    nch = n_per_w // CH

    mesh = plsc.VectorSubcoreMesh(core_axis_name="c", subcore_axis_name="s")

    @functools.partial(
        pl.kernel,
        mesh=mesh,
        out_type=jax.ShapeDtypeStruct((n, d), jnp.float32),
        compiler_params=pltpu.CompilerParams(use_tc_tiling_on_sc=False),
        scratch_types=[
            pltpu.VMEM((G, IDX_VEC), jnp.int32),
            pltpu.VMEM((CH, d), jnp.float32),
            pltpu.SemaphoreType.DMA,
        ],
    )
    def emb(idx_hbm, tbl_hbm, out_hbm, idx_v, rows_v, sem):
        wid = lax.axis_index("s") * NC + lax.axis_index("c")
        row0 = wid * (n_per_w // IDX_VEC)

        def body(c, carry):
            pltpu.sync_copy(idx_hbm.at[pl.ds(row0 + c * G, G)], idx_v)
            copies = []
            for j in range(G):
                copies.append(
                    pltpu.async_copy(
                        tbl_hbm.at[idx_v.at[j]],
                        rows_v.at[pl.ds(j * IDX_VEC, IDX_VEC)],
                        sem,
                    )
                )
            for cp in copies:
                cp.wait()
            pltpu.sync_copy(
                rows_v, out_hbm.at[pl.ds(wid * n_per_w + c * CH, CH)]
            )
            return carry

        lax.fori_loop(0, nch, body, 0)

    return emb(xf, weight)


def kernel(x, weight):
    b, h = x.shape
    v, d = weight.shape
    n = b * h
    xf = x.reshape(n // IDX_VEC, IDX_VEC)
    out = _emb_call(xf, weight, n, d)
    return out.reshape(b, h, d)


# speedup vs baseline: 1.8541x; 1.0327x over previous
import functools

import jax
import jax.numpy as jnp
from jax import lax
from jax.experimental import pallas as pl
from jax.experimental.pallas import tpu as pltpu
from jax.experimental.pallas import tpu_sc as plsc

NC = 2
NS = 16
NW = NC * NS

IDX_VEC = 128
CH = 512
G = CH // IDX_VEC


@functools.partial(jax.jit, static_argnums=(2, 3))
def _emb_call(xf, weight, n, d):
    n_per_w = n // NW
    nch = n_per_w // CH

    mesh = plsc.VectorSubcoreMesh(core_axis_name="c", subcore_axis_name="s")

    @functools.partial(
        pl.kernel,
        mesh=mesh,
        out_type=jax.ShapeDtypeStruct((n, d), jnp.float32),
        compiler_params=pltpu.CompilerParams(use_tc_tiling_on_sc=False),
        scratch_types=[
            pltpu.VMEM((2, G, IDX_VEC), jnp.int32),
            pltpu.VMEM((2, CH, d), jnp.float32),
            pltpu.SemaphoreType.DMA,
            pltpu.SemaphoreType.DMA,
            pltpu.SemaphoreType.DMA,
            pltpu.SemaphoreType.DMA,
        ],
    )
    def emb(idx_hbm, tbl_hbm, out_hbm, idx_v, rows_v, g0, g1, o0, o1):
        wid = lax.axis_index("s") * NC + lax.axis_index("c")
        row0 = wid * (n_per_w // IDX_VEC)
        obase = wid * n_per_w
        gsem = (g0, g1)
        osem = (o0, o1)

        def load_and_fire(c, b):
            pltpu.sync_copy(idx_hbm.at[pl.ds(row0 + c * G, G)], idx_v.at[b])
            for j in range(G):
                pltpu.async_copy(
                    tbl_hbm.at[idx_v.at[b, j]],
                    rows_v.at[b, pl.ds(j * IDX_VEC, IDX_VEC)],
                    gsem[b],
                )

        def wait_gathers(b):
            pltpu.make_async_copy(
                tbl_hbm.at[pl.ds(0, CH)], rows_v.at[b], gsem[b]
            ).wait()

        def fire_out(c, b):
            pltpu.async_copy(
                rows_v.at[b], out_hbm.at[pl.ds(obase + c * CH, CH)], osem[b]
            )

        def wait_out(b):
            pltpu.make_async_copy(
                rows_v.at[b], out_hbm.at[pl.ds(0, CH)], osem[b]
            ).wait()

        load_and_fire(0, 0)
        load_and_fire(1, 1)
        wait_gathers(0)
        fire_out(0, 0)

        def body(k, carry):
            c0 = 2 + 2 * k
            wait_out(0)
            load_and_fire(c0, 0)
            wait_gathers(1)
            fire_out(c0 - 1, 1)
            wait_out(1)
            load_and_fire(c0 + 1, 1)
            wait_gathers(0)
            fire_out(c0, 0)
            return carry

        lax.fori_loop(0, (nch - 2) // 2, body, 0)

        wait_out(0)
        wait_gathers(1)
        fire_out(nch - 1, 1)
        wait_out(1)

    return emb(xf, weight)


def kernel(x, weight):
    b, h = x.shape
    v, d = weight.shape
    n = b * h
    xf = x.reshape(n // IDX_VEC, IDX_VEC)
    out = _emb_call(xf, weight, n, d)
    return out.reshape(b, h, d)
